# baseline (speedup 1.0000x reference)
import functools

import jax
import jax.numpy as jnp
from jax import lax
from jax.experimental import pallas as pl
from jax.experimental.pallas import tpu as pltpu
from jax.experimental.pallas import tpu_sc as plsc

N = 10000
E = 320000
F = 128
FO = 16
NPAD = 10240
NC = 2
NS = 16
NW = NC * NS
EPW = E // NW
L = 16
SLICE = NPAD // NS
CH = 2000
BE = 40
NB = EPW // BE
ZR = 128
BM = 1024


def _mesh():
    return plsc.VectorSubcoreMesh(core_axis_name="c", subcore_axis_name="s")


_SC_PARAMS = pltpu.CompilerParams(needs_layout_passes=False)


def _zero16():
    return jnp.zeros((L,), jnp.float32)


def _fill_zero_1d(ref, n):
    z = _zero16()

    def body(i, _):
        ref[pl.ds(i * L, L)] = z
        return 0

    lax.fori_loop(0, n // L, body, 0, unroll=8)


def _fill_zero_2d(ref, rows, w):
    z = _zero16()

    def body(i, _):
        for f in range(w // L):
            ref[i, pl.ds(f * L, L)] = z
        return 0

    lax.fori_loop(0, rows, body, 0, unroll=4)


@functools.partial(
    pl.kernel,
    out_type=jax.ShapeDtypeStruct((NC, NPAD), jnp.float32),
    mesh=_mesh(),
    compiler_params=_SC_PARAMS,
    scratch_types=[
        pltpu.VMEM((CH,), jnp.int32),
        pltpu.VMEM((CH,), jnp.int32),
        pltpu.VMEM((CH,), jnp.float32),
        pltpu.VMEM((NPAD,), jnp.float32),
        pltpu.VMEM((NS, SLICE), jnp.float32),
        pltpu.VMEM_SHARED((NS, NPAD), jnp.float32),
    ],
)
def _deg(row_h, col_h, ew_h, out_h, rowb, colb, ewb, acc, red, shacc):
    c = lax.axis_index("c")
    s = lax.axis_index("s")
    wid = s * NC + c
    _fill_zero_1d(acc, NPAD)
    base = wid * EPW
    for k in range(EPW // CH):
        off = base + k * CH
        pltpu.sync_copy(row_h.at[pl.ds(off, CH)], rowb)
        pltpu.sync_copy(col_h.at[pl.ds(off, CH)], colb)
        pltpu.sync_copy(ew_h.at[pl.ds(off, CH)], ewb)

        def body(j, _):
            r = rowb[pl.ds(j * L, L)]
            cc = colb[pl.ds(j * L, L)]
            wv = jnp.where(r == cc, 0.0, ewb[pl.ds(j * L, L)])
            plsc.addupdate_scatter(acc, [r], wv)
            return 0

        lax.fori_loop(0, CH // L, body, 0)
    pltpu.sync_copy(acc, shacc.at[s])
    plsc.subcore_barrier()
    for t in range(NS):
        pltpu.sync_copy(shacc.at[t, pl.ds(s * SLICE, SLICE)], red.at[t])

    def rbody(j, _):
        v = red[0, pl.ds(j * L, L)]
        for t in range(1, NS):
            v = v + red[t, pl.ds(j * L, L)]
        acc[pl.ds(j * L, L)] = v
        return 0

    lax.fori_loop(0, SLICE // L, rbody, 0)
    pltpu.sync_copy(acc.at[pl.ds(0, SLICE)], out_h.at[c, pl.ds(s * SLICE, SLICE)])


@functools.partial(
    pl.kernel,
    out_type=jax.ShapeDtypeStruct((E,), jnp.float32),
    mesh=_mesh(),
    compiler_params=_SC_PARAMS,
    scratch_types=[
        pltpu.VMEM((NPAD,), jnp.float32),
        pltpu.VMEM((NPAD,), jnp.float32),
        pltpu.VMEM((CH,), jnp.int32),
        pltpu.VMEM((CH,), jnp.int32),
        pltpu.VMEM((CH,), jnp.float32),
        pltpu.VMEM((CH,), jnp.float32),
    ],
)
def _lapw(degp_h, row_h, col_h, ew_h, lw_h, disb, tmpb, rowb, colb, ewb, lwb):
    c = lax.axis_index("c")
    s = lax.axis_index("s")
    wid = s * NC + c
    pltpu.sync_copy(degp_h.at[0], disb)
    pltpu.sync_copy(degp_h.at[1], tmpb)

    def dbody(i, _):
        d = disb[pl.ds(i * L, L)] + tmpb[pl.ds(i * L, L)]
        ib = lax.bitcast_convert_type(d, jnp.int32)
        y = lax.bitcast_convert_type(jnp.int32(0x5F3759DF) - (ib >> 1), jnp.float32)
        y = y * (1.5 - 0.5 * d * y * y)
        y = y * (1.5 - 0.5 * d * y * y)
        y = y * (1.5 - 0.5 * d * y * y)
        disb[pl.ds(i * L, L)] = jnp.where(d > 0.0, y, 0.0)
        return 0

    lax.fori_loop(0, NPAD // L, dbody, 0, unroll=4)
    base = wid * EPW
    for k in range(EPW // CH):
        off = base + k * CH
        pltpu.sync_copy(row_h.at[pl.ds(off, CH)], rowb)
        pltpu.sync_copy(col_h.at[pl.ds(off, CH)], colb)
        pltpu.sync_copy(ew_h.at[pl.ds(off, CH)], ewb)

        def ebody(j, _):
            r = rowb[pl.ds(j * L, L)]
            cc = colb[pl.ds(j * L, L)]
            wv = jnp.where(r == cc, 0.0, ewb[pl.ds(j * L, L)])
            dr = plsc.load_gather(disb, [r])
            dc = plsc.load_gather(disb, [cc])
            lwb[pl.ds(j * L, L)] = -(dr * wv * dc)
            return 0

        lax.fori_loop(0, CH // L, ebody, 0, unroll=2)
        pltpu.sync_copy(lwb, lw_h.at[pl.ds(off, CH)])


def _make_spmv(W):
    fpr = W // L

    @functools.partial(
        pl.kernel,
        out_type=jax.ShapeDtypeStruct((NC, NPAD, W), jnp.float32),
        mesh=_mesh(),
        compiler_params=_SC_PARAMS,
        scratch_types=[
            pltpu.VMEM((EPW,), jnp.int32),
            pltpu.VMEM((EPW,), jnp.int32),
            pltpu.VMEM((EPW,), jnp.float32),
            pltpu.VMEM((3, BE, W), jnp.float32),
            pltpu.VMEM_SHARED((NPAD, W), jnp.float32),
            pltpu.SemaphoreType.DMA,
            pltpu.SemaphoreType.DMA,
            pltpu.SemaphoreType.DMA,
            pltpu.SemaphoreType.DMA,
            pltpu.SemaphoreType.DMA,
            pltpu.SemaphoreType.DMA,
        ],
    )
    def spmv(src_h, row_h, col_h, lw_h, out_h, ridx, cidx, lwb, rows, shacc,
             g0, g1, g2, s0, s1, s2):
        c = lax.axis_index("c")
        s = lax.axis_index("s")
        wid = s * NC + c
        base = wid * EPW
        gsem = (g0, g1, g2)
        ssem = (s0, s1, s2)

        pltpu.sync_copy(row_h.at[pl.ds(base, EPW)], ridx)
        pltpu.sync_copy(col_h.at[pl.ds(base, EPW)], cidx)
        pltpu.sync_copy(lw_h.at[pl.ds(base, EPW)], lwb)

        _fill_zero_2d(rows.at[0], BE, W)
        for k in range(SLICE // BE):
            pltpu.sync_copy(rows.at[0], shacc.at[pl.ds(s * SLICE + k * BE, BE)])

        def fire_g(j, b):
            pltpu.async_copy(src_h.at[cidx.at[pl.ds(j * BE, BE)]], rows.at[b], gsem[b])

        def wait_g(j, b):
            pltpu.make_async_copy(
                src_h.at[cidx.at[pl.ds(j * BE, BE)]], rows.at[b], gsem[b]
            ).wait()

        def fire_s(j, b):
            pltpu.async_copy(
                rows.at[b], shacc.at[ridx.at[pl.ds(j * BE, BE)]], ssem[b], add=True
            )

        def wait_s(j, b):
            pltpu.make_async_copy(
                rows.at[b], shacc.at[ridx.at[pl.ds(j * BE, BE)]], ssem[b]
            ).wait()

        def scale(j, b):
            jb = jnp.full((L,), j * BE, jnp.int32)

            def ebody(e, _):
                lw16 = plsc.load_gather(lwb, [jb + e])
                for f in range(fpr):
                    v = rows[b, e, pl.ds(f * L, L)]
                    rows[b, e, pl.ds(f * L, L)] = v * lw16
                return 0

            lax.fori_loop(0, BE, ebody, 0, unroll=8)

        fire_g(0, 0)
        fire_g(1, 1)
        plsc.subcore_barrier()

        def step(j, b, b2):
            wait_g(j, b)
            scale(j, b)
            fire_s(j, b)

            @pl.when(j + 2 < NB)
            def _():
                @pl.when(j >= 1)
                def _():
                    wait_s(j - 1, b2)

                fire_g(j + 2, b2)

        def outer(i, _):
            j0 = 3 * i
            step(j0, 0, 2)
            step(j0 + 1, 1, 0)
            step(j0 + 2, 2, 1)
            return 0

        lax.fori_loop(0, NB // 3, outer, 0)
        jr = NB - 1
        wait_g(jr, 0)
        scale(jr, 0)
        fire_s(jr, 0)
        wait_s(jr - 2, 1)
        wait_s(jr - 1, 2)
        wait_s(jr, 0)
        plsc.subcore_barrier()
        for k in range(SLICE // ZR):
            sl = pl.ds(s * SLICE + k * ZR, ZR)
            pltpu.sync_copy(shacc.at[sl], out_h.at[c, sl])

    return spmv


_spmv128 = _make_spmv(F)


def _c1_body(p_ref, o_ref):
    o_ref[...] = p_ref[0] + p_ref[1]


def _combine(P):
    return pl.pallas_call(
        _c1_body,
        grid=(NPAD // BM,),
        in_specs=[pl.BlockSpec((NC, BM, F), lambda i: (0, i, 0))],
        out_specs=pl.BlockSpec((BM, F), lambda i: (i, 0)),
        out_shape=jax.ShapeDtypeStruct((NPAD, F), jnp.float32),
    )(P)


def _m1_body(x_ref, tx1_ref, p2_ref, w1_ref, b1_ref, h_ref):
    xb = x_ref[...]
    tx1 = tx1_ref[...]
    tx2 = 2.0 * (p2_ref[0] + p2_ref[1]) - xb
    hb = xb @ w1_ref[0] + tx1 @ w1_ref[1] + tx2 @ w1_ref[2] + b1_ref[...]
    h_ref[...] = jnp.maximum(hb, 0.0)


def _m1(xp, tx1, P2, W1, b1):
    return pl.pallas_call(
        _m1_body,
        grid=(NPAD // BM,),
        in_specs=[
            pl.BlockSpec((BM, F), lambda i: (i, 0)),
            pl.BlockSpec((BM, F), lambda i: (i, 0)),
            pl.BlockSpec((NC, BM, F), lambda i: (0, i, 0)),
            pl.BlockSpec((3, F, F), lambda i: (0, 0, 0)),
            pl.BlockSpec((1, F), lambda i: (0, 0)),
        ],
        out_specs=pl.BlockSpec((BM, F), lambda i: (i, 0)),
        out_shape=jax.ShapeDtypeStruct((NPAD, F), jnp.float32),
    )(xp, tx1, P2, W1, b1)


def _m2_body(h_ref, sh_ref, p4_ref, w2_ref, b2_ref, o_ref):
    hb = h_ref[...]
    sh = sh_ref[...]
    tx2 = 2.0 * (p4_ref[0] + p4_ref[1]) - hb
    o_ref[...] = hb @ w2_ref[0] + sh @ w2_ref[1] + tx2 @ w2_ref[2] + b2_ref[...]


def _m2(h, sh, P4, W2, b2):
    return pl.pallas_call(
        _m2_body,
        grid=(NPAD // BM,),
        in_specs=[
            pl.BlockSpec((BM, F), lambda i: (i, 0)),
            pl.BlockSpec((BM, F), lambda i: (i, 0)),
            pl.BlockSpec((NC, BM, F), lambda i: (0, i, 0)),
            pl.BlockSpec((3, F, FO), lambda i: (0, 0, 0)),
            pl.BlockSpec((1, FO), lambda i: (0, 0)),
        ],
        out_specs=pl.BlockSpec((BM, FO), lambda i: (i, 0)),
        out_shape=jax.ShapeDtypeStruct((NPAD, FO), jnp.float32),
    )(h, sh, P4, W2, b2)


def kernel(x, edge_index, edge_weight, W1, b1, W2, b2):
    row = edge_index[0]
    col = edge_index[1]
    xp = jnp.pad(x, ((0, NPAD - N), (0, 0)))

    degp = _deg(row, col, edge_weight)
    lapw = _lapw(degp, row, col, edge_weight)

    P1 = _spmv128(xp, row, col, lapw)
    tx1 = _combine(P1)
    P2 = _spmv128(tx1, row, col, lapw)
    h = _m1(xp, tx1, P2, W1, b1.reshape(1, F))

    P3 = _spmv128(h, row, col, lapw)
    sh = _combine(P3)
    P4 = _spmv128(sh, row, col, lapw)
    out = _m2(h, sh, P4, W2, b2.reshape(1, FO))
    return out[:N]


# speedup vs baseline: 1.0031x; 1.0031x over previous
import functools

import jax
import jax.numpy as jnp
from jax import lax
from jax.experimental import pallas as pl
from jax.experimental.pallas import tpu as pltpu
from jax.experimental.pallas import tpu_sc as plsc

N = 10000
E = 320000
F = 128
FO = 16
NPAD = 10240
NC = 2
NS = 16
NW = NC * NS
EPW = E // NW
L = 16
SLICE = NPAD // NS
CH = 2000
BE = 40
NB = EPW // BE
ZR = 128
BM = 1024


def _mesh():
    return plsc.VectorSubcoreMesh(core_axis_name="c", subcore_axis_name="s")


_SC_PARAMS = pltpu.CompilerParams(needs_layout_passes=False)


def _zero16():
    return jnp.zeros((L,), jnp.float32)


def _fill_zero_1d(ref, n):
    z = _zero16()

    def body(i, _):
        ref[pl.ds(i * L, L)] = z
        return 0

    lax.fori_loop(0, n // L, body, 0, unroll=8)


def _fill_zero_2d(ref, rows, w):
    z = _zero16()

    def body(i, _):
        for f in range(w // L):
            ref[i, pl.ds(f * L, L)] = z
        return 0

    lax.fori_loop(0, rows, body, 0, unroll=4)


@functools.partial(
    pl.kernel,
    out_type=jax.ShapeDtypeStruct((NC, NPAD), jnp.float32),
    mesh=_mesh(),
    compiler_params=_SC_PARAMS,
    scratch_types=[
        pltpu.VMEM((CH,), jnp.int32),
        pltpu.VMEM((CH,), jnp.int32),
        pltpu.VMEM((CH,), jnp.float32),
        pltpu.VMEM((NPAD,), jnp.float32),
        pltpu.VMEM((NS, SLICE), jnp.float32),
        pltpu.VMEM_SHARED((NS, NPAD), jnp.float32),
    ],
)
def _deg(row_h, col_h, ew_h, out_h, rowb, colb, ewb, acc, red, shacc):
    c = lax.axis_index("c")
    s = lax.axis_index("s")
    wid = s * NC + c
    _fill_zero_1d(acc, NPAD)
    base = wid * EPW
    for k in range(EPW // CH):
        off = base + k * CH
        pltpu.sync_copy(row_h.at[pl.ds(off, CH)], rowb)
        pltpu.sync_copy(col_h.at[pl.ds(off, CH)], colb)
        pltpu.sync_copy(ew_h.at[pl.ds(off, CH)], ewb)

        def body(j, _):
            r = rowb[pl.ds(j * L, L)]
            cc = colb[pl.ds(j * L, L)]
            wv = jnp.where(r == cc, 0.0, ewb[pl.ds(j * L, L)])
            plsc.addupdate_scatter(acc, [r], wv)
            return 0

        lax.fori_loop(0, CH // L, body, 0)
    pltpu.sync_copy(acc, shacc.at[s])
    plsc.subcore_barrier()
    for t in range(NS):
        pltpu.sync_copy(shacc.at[t, pl.ds(s * SLICE, SLICE)], red.at[t])

    def rbody(j, _):
        v = red[0, pl.ds(j * L, L)]
        for t in range(1, NS):
            v = v + red[t, pl.ds(j * L, L)]
        acc[pl.ds(j * L, L)] = v
        return 0

    lax.fori_loop(0, SLICE // L, rbody, 0)
    pltpu.sync_copy(acc.at[pl.ds(0, SLICE)], out_h.at[c, pl.ds(s * SLICE, SLICE)])


@functools.partial(
    pl.kernel,
    out_type=jax.ShapeDtypeStruct((E,), jnp.float32),
    mesh=_mesh(),
    compiler_params=_SC_PARAMS,
    scratch_types=[
        pltpu.VMEM((NPAD,), jnp.float32),
        pltpu.VMEM((NPAD,), jnp.float32),
        pltpu.VMEM((CH,), jnp.int32),
        pltpu.VMEM((CH,), jnp.int32),
        pltpu.VMEM((CH,), jnp.float32),
        pltpu.VMEM((CH,), jnp.float32),
    ],
)
def _lapw(degp_h, row_h, col_h, ew_h, lw_h, disb, tmpb, rowb, colb, ewb, lwb):
    c = lax.axis_index("c")
    s = lax.axis_index("s")
    wid = s * NC + c
    pltpu.sync_copy(degp_h.at[0], disb)
    pltpu.sync_copy(degp_h.at[1], tmpb)

    def dbody(i, _):
        d = disb[pl.ds(i * L, L)] + tmpb[pl.ds(i * L, L)]
        ib = lax.bitcast_convert_type(d, jnp.int32)
        y = lax.bitcast_convert_type(jnp.int32(0x5F3759DF) - (ib >> 1), jnp.float32)
        y = y * (1.5 - 0.5 * d * y * y)
        y = y * (1.5 - 0.5 * d * y * y)
        y = y * (1.5 - 0.5 * d * y * y)
        disb[pl.ds(i * L, L)] = jnp.where(d > 0.0, y, 0.0)
        return 0

    lax.fori_loop(0, NPAD // L, dbody, 0, unroll=4)
    base = wid * EPW
    for k in range(EPW // CH):
        off = base + k * CH
        pltpu.sync_copy(row_h.at[pl.ds(off, CH)], rowb)
        pltpu.sync_copy(col_h.at[pl.ds(off, CH)], colb)
        pltpu.sync_copy(ew_h.at[pl.ds(off, CH)], ewb)

        def ebody(j, _):
            r = rowb[pl.ds(j * L, L)]
            cc = colb[pl.ds(j * L, L)]
            wv = jnp.where(r == cc, 0.0, ewb[pl.ds(j * L, L)])
            dr = plsc.load_gather(disb, [r])
            dc = plsc.load_gather(disb, [cc])
            lwb[pl.ds(j * L, L)] = -(dr * wv * dc)
            return 0

        lax.fori_loop(0, CH // L, ebody, 0, unroll=2)
        pltpu.sync_copy(lwb, lw_h.at[pl.ds(off, CH)])


def _make_spmv(W):
    fpr = W // L

    @functools.partial(
        pl.kernel,
        out_type=jax.ShapeDtypeStruct((NC, NPAD, W), jnp.float32),
        mesh=_mesh(),
        compiler_params=_SC_PARAMS,
        scratch_types=[
            pltpu.VMEM((EPW,), jnp.int32),
            pltpu.VMEM((EPW,), jnp.int32),
            pltpu.VMEM((EPW,), jnp.float32),
            pltpu.VMEM((3, BE, W), jnp.float32),
            pltpu.VMEM_SHARED((NPAD, W), jnp.float32),
            pltpu.SemaphoreType.DMA,
            pltpu.SemaphoreType.DMA,
            pltpu.SemaphoreType.DMA,
            pltpu.SemaphoreType.DMA,
            pltpu.SemaphoreType.DMA,
            pltpu.SemaphoreType.DMA,
        ],
    )
    def spmv(src_h, row_h, col_h, lw_h, out_h, ridx, cidx, lwb, rows, shacc,
             g0, g1, g2, s0, s1, s2):
        c = lax.axis_index("c")
        s = lax.axis_index("s")
        wid = s * NC + c
        base = wid * EPW
        gsem = (g0, g1, g2)
        ssem = (s0, s1, s2)

        pltpu.sync_copy(row_h.at[pl.ds(base, EPW)], ridx)
        pltpu.sync_copy(col_h.at[pl.ds(base, EPW)], cidx)
        pltpu.sync_copy(lw_h.at[pl.ds(base, EPW)], lwb)

        _fill_zero_2d(rows.at[0], BE, W)
        for k in range(SLICE // BE):
            pltpu.sync_copy(rows.at[0], shacc.at[pl.ds(s * SLICE + k * BE, BE)])

        def fire_g(j, b):
            pltpu.async_copy(src_h.at[cidx.at[pl.ds(j * BE, BE)]], rows.at[b], gsem[b])

        def wait_g(j, b):
            pltpu.make_async_copy(
                src_h.at[cidx.at[pl.ds(j * BE, BE)]], rows.at[b], gsem[b]
            ).wait()

        def fire_s(j, b):
            pltpu.async_copy(
                rows.at[b], shacc.at[ridx.at[pl.ds(j * BE, BE)]], ssem[b], add=True
            )

        def wait_s(j, b):
            pltpu.make_async_copy(
                rows.at[b], shacc.at[ridx.at[pl.ds(j * BE, BE)]], ssem[b]
            ).wait()

        def scale(j, b):
            jb = jnp.full((L,), j * BE, jnp.int32)

            def ebody(e, _):
                lw16 = plsc.load_gather(lwb, [jb + e])
                for f in range(fpr):
                    v = rows[b, e, pl.ds(f * L, L)]
                    rows[b, e, pl.ds(f * L, L)] = v * lw16
                return 0

            lax.fori_loop(0, BE, ebody, 0, unroll=4)

        fire_g(0, 0)
        fire_g(1, 1)
        plsc.subcore_barrier()

        def step(j, b, b2):
            wait_g(j, b)
            scale(j, b)
            fire_s(j, b)

            @pl.when(j + 2 < NB)
            def _():
                @pl.when(j >= 1)
                def _():
                    wait_s(j - 1, b2)

                fire_g(j + 2, b2)

        def outer(i, _):
            j0 = 3 * i
            step(j0, 0, 2)
            step(j0 + 1, 1, 0)
            step(j0 + 2, 2, 1)
            return 0

        lax.fori_loop(0, NB // 3, outer, 0)
        jr = NB - 1
        wait_g(jr, 0)
        scale(jr, 0)
        fire_s(jr, 0)
        wait_s(jr - 2, 1)
        wait_s(jr - 1, 2)
        wait_s(jr, 0)
        plsc.subcore_barrier()
        for k in range(SLICE // ZR):
            sl = pl.ds(s * SLICE + k * ZR, ZR)
            pltpu.sync_copy(shacc.at[sl], out_h.at[c, sl])

    return spmv


_spmv128 = _make_spmv(F)


def _c1_body(p_ref, o_ref):
    o_ref[...] = p_ref[0] + p_ref[1]


def _combine(P):
    return pl.pallas_call(
        _c1_body,
        grid=(NPAD // BM,),
        in_specs=[pl.BlockSpec((NC, BM, F), lambda i: (0, i, 0))],
        out_specs=pl.BlockSpec((BM, F), lambda i: (i, 0)),
        out_shape=jax.ShapeDtypeStruct((NPAD, F), jnp.float32),
    )(P)


def _m1_body(x_ref, tx1_ref, p2_ref, w1_ref, b1_ref, h_ref):
    xb = x_ref[...]
    tx1 = tx1_ref[...]
    tx2 = 2.0 * (p2_ref[0] + p2_ref[1]) - xb
    hb = xb @ w1_ref[0] + tx1 @ w1_ref[1] + tx2 @ w1_ref[2] + b1_ref[...]
    h_ref[...] = jnp.maximum(hb, 0.0)


def _m1(xp, tx1, P2, W1, b1):
    return pl.pallas_call(
        _m1_body,
        grid=(NPAD // BM,),
        in_specs=[
            pl.BlockSpec((BM, F), lambda i: (i, 0)),
            pl.BlockSpec((BM, F), lambda i: (i, 0)),
            pl.BlockSpec((NC, BM, F), lambda i: (0, i, 0)),
            pl.BlockSpec((3, F, F), lambda i: (0, 0, 0)),
            pl.BlockSpec((1, F), lambda i: (0, 0)),
        ],
        out_specs=pl.BlockSpec((BM, F), lambda i: (i, 0)),
        out_shape=jax.ShapeDtypeStruct((NPAD, F), jnp.float32),
    )(xp, tx1, P2, W1, b1)


def _m2_body(h_ref, sh_ref, p4_ref, w2_ref, b2_ref, o_ref):
    hb = h_ref[...]
    sh = sh_ref[...]
    tx2 = 2.0 * (p4_ref[0] + p4_ref[1]) - hb
    o_ref[...] = hb @ w2_ref[0] + sh @ w2_ref[1] + tx2 @ w2_ref[2] + b2_ref[...]


def _m2(h, sh, P4, W2, b2):
    return pl.pallas_call(
        _m2_body,
        grid=(NPAD // BM,),
        in_specs=[
            pl.BlockSpec((BM, F), lambda i: (i, 0)),
            pl.BlockSpec((BM, F), lambda i: (i, 0)),
            pl.BlockSpec((NC, BM, F), lambda i: (0, i, 0)),
            pl.BlockSpec((3, F, FO), lambda i: (0, 0, 0)),
            pl.BlockSpec((1, FO), lambda i: (0, 0)),
        ],
        out_specs=pl.BlockSpec((BM, FO), lambda i: (i, 0)),
        out_shape=jax.ShapeDtypeStruct((NPAD, FO), jnp.float32),
    )(h, sh, P4, W2, b2)


def kernel(x, edge_index, edge_weight, W1, b1, W2, b2):
    row = edge_index[0]
    col = edge_index[1]
    xp = jnp.pad(x, ((0, NPAD - N), (0, 0)))

    degp = _deg(row, col, edge_weight)
    lapw = _lapw(degp, row, col, edge_weight)

    P1 = _spmv128(xp, row, col, lapw)
    tx1 = _combine(P1)
    P2 = _spmv128(tx1, row, col, lapw)
    h = _m1(xp, tx1, P2, W1, b1.reshape(1, F))

    P3 = _spmv128(h, row, col, lapw)
    sh = _combine(P3)
    P4 = _spmv128(sh, row, col, lapw)
    out = _m2(h, sh, P4, W2, b2.reshape(1, FO))
    return out[:N]
